# baseline (speedup 1.0000x reference)
import functools
import math

import jax
import jax.numpy as jnp
from jax import lax
from jax.experimental import pallas as pl
from jax.experimental.pallas import tpu as pltpu
from jax.experimental.pallas import tpu_sc as plsc

S = 64.0
MARGIN = 0.5
COS_M = math.cos(MARGIN)
SIN_M = math.sin(MARGIN)
THETA = math.cos(math.pi - MARGIN)
SINMM = math.sin(math.pi - MARGIN) * MARGIN

B = 1024
N = 100000

_NC = 2
_NS = 16
_NW = _NC * _NS
_RPW = B // _NW
_L = 16


def _sc_margin_body(flat_hbm, labels_hbm, out_hbm, lab_v, idx_v, val_v, fin_v, sem):
    wid = lax.axis_index("s") * _NC + lax.axis_index("c")
    base = wid * _RPW
    pltpu.sync_copy(labels_hbm.at[pl.ds(base, _RPW)], lab_v)
    for c in range(_RPW // _L):
        lab = lab_v[pl.ds(c * _L, _L)]
        safe = jnp.maximum(lab, 0)
        rows = base + c * _L + lax.broadcasted_iota(jnp.int32, (_L,), 0)
        idx_v[pl.ds(c * _L, _L)] = rows * N + safe
    pltpu.async_copy(flat_hbm.at[idx_v], val_v, sem).wait()
    for c in range(_RPW // _L):
        t = val_v[pl.ds(c * _L, _L)]
        u = 1.0 - t * t
        i = lax.bitcast_convert_type(u, jnp.int32)
        i = 0x5F3759DF - lax.shift_right_logical(i, 1)
        y = lax.bitcast_convert_type(i, jnp.float32)
        for _ in range(3):
            y = y * (1.5 - 0.5 * u * y * y)
        sin_t = u * y
        cosm = t * COS_M - sin_t * SIN_M
        fin = jnp.where(t > THETA, cosm, t - SINMM)
        fin_v[pl.ds(c * _L, _L)] = fin
    pltpu.sync_copy(fin_v, out_hbm.at[pl.ds(base, _RPW)])


@functools.cache
def _sc_margin():
    return pl.kernel(
        _sc_margin_body,
        mesh=plsc.VectorSubcoreMesh(core_axis_name="c", subcore_axis_name="s"),
        out_type=jax.ShapeDtypeStruct((B,), jnp.float32),
        scratch_types=[
            pltpu.VMEM((_RPW,), jnp.int32),
            pltpu.VMEM((_RPW,), jnp.int32),
            pltpu.VMEM((_RPW,), jnp.float32),
            pltpu.VMEM((_RPW,), jnp.float32),
            pltpu.SemaphoreType.DMA,
        ],
    )


_RB = 16


def _tc_body(lab_ref, fin_ref, x_ref, o_ref):
    x = x_ref[...]
    col = lax.broadcasted_iota(jnp.int32, x.shape, 1)
    mask = col == lab_ref[...]
    o_ref[...] = jnp.where(mask, fin_ref[...], x) * S


def _tc_scale_merge(logits, labels2d, fin2d):
    grid = (B // _RB,)
    return pl.pallas_call(
        _tc_body,
        grid=grid,
        in_specs=[
            pl.BlockSpec((_RB, 1), lambda i: (i, 0)),
            pl.BlockSpec((_RB, 1), lambda i: (i, 0)),
            pl.BlockSpec((_RB, N), lambda i: (i, 0)),
        ],
        out_specs=pl.BlockSpec((_RB, N), lambda i: (i, 0)),
        out_shape=jax.ShapeDtypeStruct((B, N), jnp.float32),
    )(labels2d, fin2d, logits)


@jax.jit
def kernel(logits, labels):
    labels = labels.astype(jnp.int32)
    finalv = _sc_margin()(logits.reshape(-1), labels)
    return _tc_scale_merge(logits, labels.reshape(B, 1), finalv.reshape(B, 1))


# speedup vs baseline: 1.5781x; 1.5781x over previous
import functools
import math

import jax
import jax.numpy as jnp
from jax import lax
from jax.experimental import pallas as pl
from jax.experimental.pallas import tpu as pltpu
from jax.experimental.pallas import tpu_sc as plsc

S = 64.0
MARGIN = 0.5
COS_M = math.cos(MARGIN)
SIN_M = math.sin(MARGIN)
THETA = math.cos(math.pi - MARGIN)
SINMM = math.sin(math.pi - MARGIN) * MARGIN

B = 1024
N = 100000

_NC = 2
_NS = 16
_NW = _NC * _NS
_RPW = B // _NW
_L = 16


def _sc_margin_body(flat_hbm, labels_hbm, out_hbm, lab_v, idx_v, val_v, fin_v, sem):
    wid = lax.axis_index("s") * _NC + lax.axis_index("c")
    base = wid * _RPW
    pltpu.sync_copy(labels_hbm.at[pl.ds(base, _RPW)], lab_v)
    for c in range(_RPW // _L):
        lab = lab_v[pl.ds(c * _L, _L)]
        safe = jnp.maximum(lab, 0)
        rows = base + c * _L + lax.broadcasted_iota(jnp.int32, (_L,), 0)
        idx_v[pl.ds(c * _L, _L)] = rows * N + safe
    pltpu.async_copy(flat_hbm.at[idx_v], val_v, sem).wait()
    for c in range(_RPW // _L):
        t = val_v[pl.ds(c * _L, _L)]
        u = 1.0 - t * t
        i = lax.bitcast_convert_type(u, jnp.int32)
        i = 0x5F3759DF - lax.shift_right_logical(i, 1)
        y = lax.bitcast_convert_type(i, jnp.float32)
        for _ in range(3):
            y = y * (1.5 - 0.5 * u * y * y)
        sin_t = u * y
        cosm = t * COS_M - sin_t * SIN_M
        fin = jnp.where(t > THETA, cosm, t - SINMM)
        fin_v[pl.ds(c * _L, _L)] = fin
    pltpu.sync_copy(fin_v, out_hbm.at[pl.ds(base, _RPW)])


@functools.cache
def _sc_margin():
    return pl.kernel(
        _sc_margin_body,
        mesh=plsc.VectorSubcoreMesh(core_axis_name="c", subcore_axis_name="s"),
        out_type=jax.ShapeDtypeStruct((B,), jnp.float32),
        scratch_types=[
            pltpu.VMEM((_RPW,), jnp.int32),
            pltpu.VMEM((_RPW,), jnp.int32),
            pltpu.VMEM((_RPW,), jnp.float32),
            pltpu.VMEM((_RPW,), jnp.float32),
            pltpu.SemaphoreType.DMA,
        ],
    )


_RB = 16


def _tc_body(lab_ref, fin_ref, x_ref, o_ref):
    x = x_ref[...]
    col = lax.broadcasted_iota(jnp.int32, x.shape, 1)
    mask = col == lab_ref[...]
    o_ref[...] = jnp.where(mask, fin_ref[...], x) * S


def _tc_scale_merge(logits, labels2d, fin2d):
    grid = (B // _RB,)
    return pl.pallas_call(
        _tc_body,
        grid=grid,
        in_specs=[
            pl.BlockSpec((_RB, 1), lambda i: (i, 0)),
            pl.BlockSpec((_RB, 1), lambda i: (i, 0)),
            pl.BlockSpec((_RB, N), lambda i: (i, 0)),
        ],
        out_specs=pl.BlockSpec((_RB, N), lambda i: (i, 0)),
        out_shape=jax.ShapeDtypeStruct((B, N), jnp.float32),
    )(labels2d, fin2d, logits)


@jax.jit
def kernel(logits, labels):
    labels = labels.astype(jnp.int32)
    tgt = logits[jnp.arange(B), jnp.maximum(labels, 0)]
    sin_t = jnp.sqrt(1.0 - tgt * tgt)
    finalv = jnp.where(tgt > THETA, tgt * COS_M - sin_t * SIN_M, tgt - SINMM)
    return _tc_scale_merge(logits, labels.reshape(B, 1), finalv.reshape(B, 1))


# speedup vs baseline: 1.6038x; 1.0163x over previous
import functools
import math

import jax
import jax.numpy as jnp
from jax import lax
from jax.experimental import pallas as pl
from jax.experimental.pallas import tpu as pltpu
from jax.experimental.pallas import tpu_sc as plsc

S = 64.0
MARGIN = 0.5
COS_M = math.cos(MARGIN)
SIN_M = math.sin(MARGIN)
THETA = math.cos(math.pi - MARGIN)
SINMM = math.sin(math.pi - MARGIN) * MARGIN

B = 1024
N = 100000

_NC = 2
_NS = 16
_NW = _NC * _NS
_RPW = B // _NW
_L = 16


def _sc_margin_body(flat_hbm, labels_hbm, out_hbm, lab_v, idx_v, val_v, fin_v, sem):
    wid = lax.axis_index("s") * _NC + lax.axis_index("c")
    base = wid * _RPW
    pltpu.sync_copy(labels_hbm.at[pl.ds(base, _RPW)], lab_v)
    for c in range(_RPW // _L):
        lab = lab_v[pl.ds(c * _L, _L)]
        safe = jnp.maximum(lab, 0)
        rows = base + c * _L + lax.broadcasted_iota(jnp.int32, (_L,), 0)
        idx_v[pl.ds(c * _L, _L)] = rows * N + safe
    pltpu.async_copy(flat_hbm.at[idx_v], val_v, sem).wait()
    for c in range(_RPW // _L):
        t = val_v[pl.ds(c * _L, _L)]
        u = 1.0 - t * t
        i = lax.bitcast_convert_type(u, jnp.int32)
        i = 0x5F3759DF - lax.shift_right_logical(i, 1)
        y = lax.bitcast_convert_type(i, jnp.float32)
        for _ in range(3):
            y = y * (1.5 - 0.5 * u * y * y)
        sin_t = u * y
        cosm = t * COS_M - sin_t * SIN_M
        fin = jnp.where(t > THETA, cosm, t - SINMM)
        fin_v[pl.ds(c * _L, _L)] = fin
    pltpu.sync_copy(fin_v, out_hbm.at[pl.ds(base, _RPW)])


@functools.cache
def _sc_margin():
    return pl.kernel(
        _sc_margin_body,
        mesh=plsc.VectorSubcoreMesh(core_axis_name="c", subcore_axis_name="s"),
        out_type=jax.ShapeDtypeStruct((B,), jnp.float32),
        scratch_types=[
            pltpu.VMEM((_RPW,), jnp.int32),
            pltpu.VMEM((_RPW,), jnp.int32),
            pltpu.VMEM((_RPW,), jnp.float32),
            pltpu.VMEM((_RPW,), jnp.float32),
            pltpu.SemaphoreType.DMA,
        ],
    )


_RB = 16


def _tc_body(lab_ref, x_ref, o_ref):
    x = x_ref[...]
    col = lax.broadcasted_iota(jnp.int32, x.shape, 1)
    mask = col == lab_ref[...]
    tgt = jnp.sum(jnp.where(mask, x, 0.0), axis=1, keepdims=True)
    sin_t = jnp.sqrt(1.0 - tgt * tgt)
    cosm = tgt * COS_M - sin_t * SIN_M
    fin = jnp.where(tgt > THETA, cosm, tgt - SINMM)
    o_ref[...] = jnp.where(mask, fin, x) * S


def _tc_scale_merge(logits, labels2d):
    grid = (B // _RB,)
    return pl.pallas_call(
        _tc_body,
        grid=grid,
        in_specs=[
            pl.BlockSpec((_RB, 1), lambda i: (i, 0)),
            pl.BlockSpec((_RB, N), lambda i: (i, 0)),
        ],
        out_specs=pl.BlockSpec((_RB, N), lambda i: (i, 0)),
        out_shape=jax.ShapeDtypeStruct((B, N), jnp.float32),
    )(labels2d, logits)


@jax.jit
def kernel(logits, labels):
    labels = labels.astype(jnp.int32)
    return _tc_scale_merge(logits, labels.reshape(B, 1))
